# Initial kernel scaffold; baseline (speedup 1.0000x reference)
import jax
import jax.numpy as jnp
from jax.experimental import pallas as pl


def kernel(z, scaffold, W1_0, W2_0, Wg_0, We_0, W1_1, W2_1, Wge_1, Wgs_1, We1_1, We2_1, W1_2, W2_2, Wge_2, Wgs_2, We1_2, We2_2):
    raise NotImplementedError("write your pallas kernel here")


# speedup vs baseline: 1.3693x; 1.3693x over previous
import jax
import jax.numpy as jnp
from jax.experimental import pallas as pl
from jax.experimental.pallas import tpu as pltpu

B, N, D, E = 8, 512, 32, 8
IB = 64


def _compute_m(m_ref, scaf_ref, edges_ref, Wg_s, Wg_e, h2t):
    use_edges = Wg_e is not None

    def blk(ib, carry):
        i0 = ib * IB
        a = [scaf_ref[0, e, pl.ds(i0, IB), :] for e in range(E)]
        if use_edges:
            eg = [edges_ref[e, pl.ds(i0, IB), :] for e in range(E)]
        cols = []
        for d in range(D):
            g = a[0] * Wg_s[0, d]
            for e in range(1, E):
                g += a[e] * Wg_s[e, d]
            if use_edges:
                for e in range(E):
                    g += eg[e] * Wg_e[e, d]
            g = jnp.maximum(g, 0.0)
            t = g * h2t[d, :][None, :]
            cols.append(jnp.sum(t, axis=1, keepdims=True))
        m_ref[pl.ds(i0, IB), :] = jnp.concatenate(cols, axis=1)
        return carry

    jax.lax.fori_loop(0, N // IB, blk, 0)


def _edges_init(edges_ref, scaf_ref, We_s):
    def blk(ib, carry):
        i0 = ib * IB
        a = [scaf_ref[0, e, pl.ds(i0, IB), :] for e in range(E)]
        for f in range(E):
            g = a[0] * We_s[0, f]
            for e in range(1, E):
                g += a[e] * We_s[e, f]
            edges_ref[f, pl.ds(i0, IB), :] = jnp.maximum(g, 0.0)
        return carry

    jax.lax.fori_loop(0, N // IB, blk, 0)


def _edges_update(edges_ref, scaf_ref, We1_s, We2_s):
    def blk(ib, carry):
        i0 = ib * IB
        a = [scaf_ref[0, e, pl.ds(i0, IB), :] for e in range(E)]
        eg = [edges_ref[e, pl.ds(i0, IB), :] for e in range(E)]
        for f in range(E):
            g = eg[0] * We1_s[0, f] + a[0] * We2_s[0, f]
            for e in range(1, E):
                g += eg[e] * We1_s[e, f] + a[e] * We2_s[e, f]
            edges_ref[f, pl.ds(i0, IB), :] = eg[f] + jnp.maximum(g, 0.0)
        return carry

    jax.lax.fori_loop(0, N // IB, blk, 0)


def _body(scaf_ref, z_ref,
          W1_0r, W2_0r, Wg_0s, We_0s,
          W1_1r, W2_1r, Wge_1s, Wgs_1s, We1_1s, We2_1s,
          W1_2r, W2_2r, Wge_2s, Wgs_2s, We1_2s, We2_2s,
          x_out_ref, edges_ref, m_ref):
    z = z_ref[0]
    h2t = jnp.dot(z, W2_0r[...], preferred_element_type=jnp.float32).T
    _compute_m(m_ref, scaf_ref, None, Wg_0s, None, h2t)
    x = jnp.maximum(jnp.dot(z, W1_0r[...], preferred_element_type=jnp.float32) + m_ref[...], 0.0)
    _edges_init(edges_ref, scaf_ref, We_0s)
    h2t = jnp.dot(x, W2_1r[...], preferred_element_type=jnp.float32).T
    _compute_m(m_ref, scaf_ref, edges_ref, Wgs_1s, Wge_1s, h2t)
    x = x + jnp.maximum(jnp.dot(x, W1_1r[...], preferred_element_type=jnp.float32) + m_ref[...], 0.0)
    _edges_update(edges_ref, scaf_ref, We1_1s, We2_1s)
    h2t = jnp.dot(x, W2_2r[...], preferred_element_type=jnp.float32).T
    _compute_m(m_ref, scaf_ref, edges_ref, Wgs_2s, Wge_2s, h2t)
    x_out_ref[0] = jnp.maximum(
        jnp.dot(x, W1_2r[...], preferred_element_type=jnp.float32) + m_ref[...], 0.0)


def kernel(z, scaffold, W1_0, W2_0, Wg_0, We_0,
           W1_1, W2_1, Wge_1, Wgs_1, We1_1, We2_1,
           W1_2, W2_2, Wge_2, Wgs_2, We1_2, We2_2):
    smem_spec = pl.BlockSpec(memory_space=pltpu.SMEM)
    dd_spec = pl.BlockSpec((D, D), lambda b: (0, 0))
    in_specs = [
        pl.BlockSpec((1, E, N, N), lambda b: (b, 0, 0, 0)),
        pl.BlockSpec((1, N, D), lambda b: (b, 0, 0)),
        dd_spec, dd_spec, smem_spec, smem_spec,
        dd_spec, dd_spec, smem_spec, smem_spec, smem_spec, smem_spec,
        dd_spec, dd_spec, smem_spec, smem_spec, smem_spec, smem_spec,
    ]
    grid = (B,)
    out = pl.pallas_call(
        _body,
        grid=grid,
        in_specs=in_specs,
        out_specs=pl.BlockSpec((1, N, D), lambda b: (b, 0, 0)),
        out_shape=jax.ShapeDtypeStruct((B, N, D), jnp.float32),
        scratch_shapes=[
            pltpu.VMEM((E, N, N), jnp.float32),
            pltpu.VMEM((N, D), jnp.float32),
        ],
        compiler_params=pltpu.CompilerParams(
            dimension_semantics=("parallel",),
        ),
    )(scaffold, z, W1_0, W2_0, Wg_0, We_0,
      W1_1, W2_1, Wge_1, Wgs_1, We1_1, We2_1,
      W1_2, W2_2, Wge_2, Wgs_2, We1_2, We2_2)
    return out


# speedup vs baseline: 2.1135x; 1.5435x over previous
import jax
import jax.numpy as jnp
from jax.experimental import pallas as pl
from jax.experimental.pallas import tpu as pltpu

B, N, D, E = 8, 512, 32, 8
IB = 64
BF = jnp.bfloat16


def _compute_m(m_ref, scaf_ref, edges_ref, Wg_s, Wg_e, h2t):
    use_edges = Wg_e is not None

    def blk(ib, carry):
        i0 = ib * IB
        a = [scaf_ref[0, e, pl.ds(i0, IB), :] for e in range(E)]
        if use_edges:
            eg = [edges_ref[e, pl.ds(i0, IB), :] for e in range(E)]
        cols = []
        for d in range(D):
            g = a[0] * Wg_s[0, d].astype(BF)
            for e in range(1, E):
                g += a[e] * Wg_s[e, d].astype(BF)
            if use_edges:
                for e in range(E):
                    g += eg[e] * Wg_e[e, d].astype(BF)
            g = jnp.maximum(g, BF(0.0))
            t = g.astype(jnp.float32) * h2t[d, :][None, :]
            cols.append(jnp.sum(t, axis=1, keepdims=True))
        m_ref[pl.ds(i0, IB), :] = jnp.concatenate(cols, axis=1)
        return carry

    jax.lax.fori_loop(0, N // IB, blk, 0)


def _edges_init(edges_ref, scaf_ref, We_s):
    def blk(ib, carry):
        i0 = ib * IB
        a = [scaf_ref[0, e, pl.ds(i0, IB), :] for e in range(E)]
        for f in range(E):
            g = a[0] * We_s[0, f].astype(BF)
            for e in range(1, E):
                g += a[e] * We_s[e, f].astype(BF)
            edges_ref[f, pl.ds(i0, IB), :] = jnp.maximum(g, BF(0.0))
        return carry

    jax.lax.fori_loop(0, N // IB, blk, 0)


def _edges_update(edges_ref, scaf_ref, We1_s, We2_s):
    def blk(ib, carry):
        i0 = ib * IB
        a = [scaf_ref[0, e, pl.ds(i0, IB), :] for e in range(E)]
        eg = [edges_ref[e, pl.ds(i0, IB), :] for e in range(E)]
        for f in range(E):
            g = eg[0] * We1_s[0, f].astype(BF) + a[0] * We2_s[0, f].astype(BF)
            for e in range(1, E):
                g += eg[e] * We1_s[e, f].astype(BF) + a[e] * We2_s[e, f].astype(BF)
            edges_ref[f, pl.ds(i0, IB), :] = eg[f] + jnp.maximum(g, BF(0.0))
        return carry

    jax.lax.fori_loop(0, N // IB, blk, 0)


def _body(scaf_ref, z_ref,
          W1_0r, W2_0r, Wg_0s, We_0s,
          W1_1r, W2_1r, Wge_1s, Wgs_1s, We1_1s, We2_1s,
          W1_2r, W2_2r, Wge_2s, Wgs_2s, We1_2s, We2_2s,
          x_out_ref, edges_ref, m_ref):
    z = z_ref[0]
    h2t = jnp.dot(z, W2_0r[...], preferred_element_type=jnp.float32).T
    _compute_m(m_ref, scaf_ref, None, Wg_0s, None, h2t)
    x = jnp.maximum(jnp.dot(z, W1_0r[...], preferred_element_type=jnp.float32) + m_ref[...], 0.0)
    _edges_init(edges_ref, scaf_ref, We_0s)
    h2t = jnp.dot(x, W2_1r[...], preferred_element_type=jnp.float32).T
    _compute_m(m_ref, scaf_ref, edges_ref, Wgs_1s, Wge_1s, h2t)
    x = x + jnp.maximum(jnp.dot(x, W1_1r[...], preferred_element_type=jnp.float32) + m_ref[...], 0.0)
    _edges_update(edges_ref, scaf_ref, We1_1s, We2_1s)
    h2t = jnp.dot(x, W2_2r[...], preferred_element_type=jnp.float32).T
    _compute_m(m_ref, scaf_ref, edges_ref, Wgs_2s, Wge_2s, h2t)
    x_out_ref[0] = jnp.maximum(
        jnp.dot(x, W1_2r[...], preferred_element_type=jnp.float32) + m_ref[...], 0.0)


def kernel(z, scaffold, W1_0, W2_0, Wg_0, We_0,
           W1_1, W2_1, Wge_1, Wgs_1, We1_1, We2_1,
           W1_2, W2_2, Wge_2, Wgs_2, We1_2, We2_2):
    scaffold_bf = scaffold.astype(BF)
    smem_spec = pl.BlockSpec(memory_space=pltpu.SMEM)
    dd_spec = pl.BlockSpec((D, D), lambda b: (0, 0))
    in_specs = [
        pl.BlockSpec((1, E, N, N), lambda b: (b, 0, 0, 0)),
        pl.BlockSpec((1, N, D), lambda b: (b, 0, 0)),
        dd_spec, dd_spec, smem_spec, smem_spec,
        dd_spec, dd_spec, smem_spec, smem_spec, smem_spec, smem_spec,
        dd_spec, dd_spec, smem_spec, smem_spec, smem_spec, smem_spec,
    ]
    grid = (B,)
    out = pl.pallas_call(
        _body,
        grid=grid,
        in_specs=in_specs,
        out_specs=pl.BlockSpec((1, N, D), lambda b: (b, 0, 0)),
        out_shape=jax.ShapeDtypeStruct((B, N, D), jnp.float32),
        scratch_shapes=[
            pltpu.VMEM((E, N, N), BF),
            pltpu.VMEM((N, D), jnp.float32),
        ],
        compiler_params=pltpu.CompilerParams(
            dimension_semantics=("parallel",),
        ),
    )(scaffold_bf, z, W1_0, W2_0, Wg_0, We_0,
      W1_1, W2_1, Wge_1, Wgs_1, We1_1, We2_1,
      W1_2, W2_2, Wge_2, Wgs_2, We1_2, We2_2)
    return out


# speedup vs baseline: 2.5458x; 1.2045x over previous
import jax
import jax.numpy as jnp
from jax.experimental import pallas as pl
from jax.experimental.pallas import tpu as pltpu

B, N, D, E = 8, 512, 32, 8
NN = N * N
LB = 32
L = LB * N
NBLK = N // LB
BF = jnp.bfloat16


def _layer_pass(X_ref, mT_ref, Wm_ref, h2rep, k_rows, write_edges, residual_edges):
    def dot_blk(bi):
        Xb = X_ref[0:k_rows, pl.ds(bi * L, L)]
        return jnp.dot(Wm_ref[...], Xb, preferred_element_type=jnp.float32)

    def consume(bi, G):
        t = jnp.maximum(G[0:D], 0.0) * h2rep
        cols = [jnp.sum(t[:, i * N:(i + 1) * N], axis=1, keepdims=True)
                for i in range(LB)]
        mT_ref[pl.ds(bi, 1)] = jnp.concatenate(cols, axis=1)[None]
        if write_edges:
            eg = jnp.maximum(G[D:D + E], 0.0).astype(BF)
            if residual_edges:
                eg = X_ref[E:2 * E, pl.ds(bi * L, L)] + eg
            X_ref[E:2 * E, pl.ds(bi * L, L)] = eg

    def body(bi, Gprev):
        Gnew = dot_blk(bi)
        consume(bi - 1, Gprev)
        return Gnew

    Glast = jax.lax.fori_loop(1, NBLK, body, dot_blk(0))
    consume(NBLK - 1, Glast)


def _read_m(mT_ref):
    return jnp.concatenate([mT_ref[bi] for bi in range(NBLK)], axis=1)


def _body(scaff_ref, zT_ref,
          W1T0r, W2T0r, Wm0r,
          W1T1r, W2T1r, Wm1r,
          W1T2r, W2T2r, Wm2r,
          xT_out_ref, X_ref, mT_ref):
    X_ref[0:E, :] = scaff_ref[0]
    xT = zT_ref[0]
    h2T = jnp.dot(W2T0r[...], xT, preferred_element_type=jnp.float32)
    h2rep = jnp.concatenate([h2T] * LB, axis=1)
    _layer_pass(X_ref, mT_ref, Wm0r, h2rep, E, True, False)
    xT = jnp.maximum(jnp.dot(W1T0r[...], xT, preferred_element_type=jnp.float32)
                     + _read_m(mT_ref), 0.0)
    h2T = jnp.dot(W2T1r[...], xT, preferred_element_type=jnp.float32)
    h2rep = jnp.concatenate([h2T] * LB, axis=1)
    _layer_pass(X_ref, mT_ref, Wm1r, h2rep, 2 * E, True, True)
    xT = xT + jnp.maximum(jnp.dot(W1T1r[...], xT, preferred_element_type=jnp.float32)
                          + _read_m(mT_ref), 0.0)
    h2T = jnp.dot(W2T2r[...], xT, preferred_element_type=jnp.float32)
    h2rep = jnp.concatenate([h2T] * LB, axis=1)
    _layer_pass(X_ref, mT_ref, Wm2r, h2rep, 2 * E, False, False)
    xT_out_ref[0] = jnp.maximum(
        jnp.dot(W1T2r[...], xT, preferred_element_type=jnp.float32) + _read_m(mT_ref), 0.0)


def kernel(z, scaffold, W1_0, W2_0, Wg_0, We_0,
           W1_1, W2_1, Wge_1, Wgs_1, We1_1, We2_1,
           W1_2, W2_2, Wge_2, Wgs_2, We1_2, We2_2):
    scaff = scaffold.reshape(B, E, NN).astype(BF)
    zT = jnp.transpose(z, (0, 2, 1))
    Wm0 = jnp.concatenate([Wg_0.T, We_0.T], axis=0).astype(BF)
    Wm1 = jnp.concatenate(
        [jnp.concatenate([Wgs_1.T, Wge_1.T], axis=1),
         jnp.concatenate([We2_1.T, We1_1.T], axis=1)], axis=0).astype(BF)
    Wm2 = jnp.concatenate([Wgs_2.T, Wge_2.T], axis=1).astype(BF)

    full = lambda s: pl.BlockSpec(s, lambda b: tuple(0 for _ in s))
    in_specs = [
        pl.BlockSpec((1, E, NN), lambda b: (b, 0, 0)),
        pl.BlockSpec((1, D, N), lambda b: (b, 0, 0)),
        full((D, D)), full((D, D)), full((D + E, E)),
        full((D, D)), full((D, D)), full((D + E, 2 * E)),
        full((D, D)), full((D, D)), full((D, 2 * E)),
    ]
    out = pl.pallas_call(
        _body,
        grid=(B,),
        in_specs=in_specs,
        out_specs=pl.BlockSpec((1, D, N), lambda b: (b, 0, 0)),
        out_shape=jax.ShapeDtypeStruct((B, D, N), jnp.float32),
        scratch_shapes=[
            pltpu.VMEM((2 * E, NN), BF),
            pltpu.VMEM((NBLK, D, LB), jnp.float32),
        ],
        compiler_params=pltpu.CompilerParams(
            dimension_semantics=("parallel",),
        ),
    )(scaff, zT, W1_0.T, W2_0.T, Wm0,
      W1_1.T, W2_1.T, Wm1,
      W1_2.T, W2_2.T, Wm2)
    return jnp.transpose(out, (0, 2, 1))


# speedup vs baseline: 2.6068x; 1.0240x over previous
import jax
import jax.numpy as jnp
from jax.experimental import pallas as pl
from jax.experimental.pallas import tpu as pltpu

B, N, D, E = 8, 512, 32, 8
NN = N * N
LB = 64
L = LB * N
NBLK = N // LB
BF = jnp.bfloat16


def _layer_pass(X_ref, mT_ref, Wm_ref, h2Tb, k_rows, write_edges, residual_edges):
    def dot_blk(bi):
        Xb = X_ref[0:k_rows, pl.ds(bi * L, L)]
        return jnp.dot(Wm_ref[...], Xb, preferred_element_type=jnp.float32)

    def consume(bi, G):
        cols = [jnp.sum(jnp.maximum(G[0:D, i * N:(i + 1) * N], 0.0) * h2Tb,
                        axis=1, keepdims=True)
                for i in range(LB)]
        mT_ref[pl.ds(bi, 1)] = jnp.concatenate(cols, axis=1)[None]
        if write_edges:
            eg = jnp.maximum(G[D:D + E], 0.0).astype(BF)
            if residual_edges:
                eg = X_ref[E:2 * E, pl.ds(bi * L, L)] + eg
            X_ref[E:2 * E, pl.ds(bi * L, L)] = eg

    def body(bi, Gprev):
        Gnew = dot_blk(bi)
        consume(bi - 1, Gprev)
        return Gnew

    Glast = jax.lax.fori_loop(1, NBLK, body, dot_blk(0))
    consume(NBLK - 1, Glast)


def _read_m(mT_ref):
    return jnp.concatenate([mT_ref[bi] for bi in range(NBLK)], axis=1)


def _body(scaff_ref, zT_ref,
          W1T0r, W2T0r, Wm0r,
          W1T1r, W2T1r, Wm1r,
          W1T2r, W2T2r, Wm2r,
          xT_out_ref, X_ref, mT_ref):
    X_ref[0:E, :] = scaff_ref[0]
    xT = zT_ref[0]
    h2Tb = jnp.dot(W2T0r[...], xT, preferred_element_type=jnp.float32)
    _layer_pass(X_ref, mT_ref, Wm0r, h2Tb, E, True, False)
    xT = jnp.maximum(jnp.dot(W1T0r[...], xT, preferred_element_type=jnp.float32)
                     + _read_m(mT_ref), 0.0)
    h2Tb = jnp.dot(W2T1r[...], xT, preferred_element_type=jnp.float32)
    _layer_pass(X_ref, mT_ref, Wm1r, h2Tb, 2 * E, True, True)
    xT = xT + jnp.maximum(jnp.dot(W1T1r[...], xT, preferred_element_type=jnp.float32)
                          + _read_m(mT_ref), 0.0)
    h2Tb = jnp.dot(W2T2r[...], xT, preferred_element_type=jnp.float32)
    _layer_pass(X_ref, mT_ref, Wm2r, h2Tb, 2 * E, False, False)
    xT_out_ref[0] = jnp.maximum(
        jnp.dot(W1T2r[...], xT, preferred_element_type=jnp.float32) + _read_m(mT_ref), 0.0)


def kernel(z, scaffold, W1_0, W2_0, Wg_0, We_0,
           W1_1, W2_1, Wge_1, Wgs_1, We1_1, We2_1,
           W1_2, W2_2, Wge_2, Wgs_2, We1_2, We2_2):
    scaff = scaffold.reshape(B, E, NN).astype(BF)
    zT = jnp.transpose(z, (0, 2, 1))
    Wm0 = jnp.concatenate([Wg_0.T, We_0.T], axis=0).astype(BF)
    Wm1 = jnp.concatenate(
        [jnp.concatenate([Wgs_1.T, Wge_1.T], axis=1),
         jnp.concatenate([We2_1.T, We1_1.T], axis=1)], axis=0).astype(BF)
    Wm2 = jnp.concatenate([Wgs_2.T, Wge_2.T], axis=1).astype(BF)

    full = lambda s: pl.BlockSpec(s, lambda b: tuple(0 for _ in s))
    in_specs = [
        pl.BlockSpec((1, E, NN), lambda b: (b, 0, 0)),
        pl.BlockSpec((1, D, N), lambda b: (b, 0, 0)),
        full((D, D)), full((D, D)), full((D + E, E)),
        full((D, D)), full((D, D)), full((D + E, 2 * E)),
        full((D, D)), full((D, D)), full((D, 2 * E)),
    ]
    out = pl.pallas_call(
        _body,
        grid=(B,),
        in_specs=in_specs,
        out_specs=pl.BlockSpec((1, D, N), lambda b: (b, 0, 0)),
        out_shape=jax.ShapeDtypeStruct((B, D, N), jnp.float32),
        scratch_shapes=[
            pltpu.VMEM((2 * E, NN), BF),
            pltpu.VMEM((NBLK, D, LB), jnp.float32),
        ],
        compiler_params=pltpu.CompilerParams(
            dimension_semantics=("parallel",),
        ),
    )(scaff, zT, W1_0.T, W2_0.T, Wm0,
      W1_1.T, W2_1.T, Wm1,
      W1_2.T, W2_2.T, Wm2)
    return jnp.transpose(out, (0, 2, 1))
